# Initial kernel scaffold; baseline (speedup 1.0000x reference)
import functools

import jax
import jax.numpy as jnp
from jax import lax
from jax.experimental import pallas as pl
from jax.experimental.pallas import tpu as pltpu
from jax.experimental.pallas import tpu_sc as plsc

H = 64
SUB = 128
KS = (9, 70, 200, 50)


@functools.lru_cache(maxsize=None)
def _make_embed_call(batch_size):
    info = plsc.get_sparse_core_info()
    nc, ns = info.num_cores, info.num_subcores
    nw = nc * ns
    rows_per_worker = batch_size // nw
    assert rows_per_worker % SUB == 0
    sub_groups = rows_per_worker // SUB

    mesh = plsc.VectorSubcoreMesh(core_axis_name="c", subcore_axis_name="s")

    out_type = tuple(
        jax.ShapeDtypeStruct((batch_size * k, H), jnp.float32) for k in KS
    )

    @functools.partial(
        pl.kernel,
        mesh=mesh,
        out_type=out_type,
        scratch_types=[
            pltpu.VMEM((SUB,), jnp.int32),
            pltpu.VMEM((SUB, H), jnp.float32),
            pltpu.SemaphoreType.DMA,
        ],
    )
    def embed(demo_i, vital_i, dx_i, proc_i,
              demo_t, vital_t, dx_t, proc_t,
              demo_o, vital_o, dx_o, proc_o,
              idx_v, rows_v, sem):
        wid = lax.axis_index("s") * nc + lax.axis_index("c")
        for (idx_hbm, tbl, out_hbm, k) in (
            (demo_i, demo_t, demo_o, KS[0]),
            (vital_i, vital_t, vital_o, KS[1]),
            (dx_i, dx_t, dx_o, KS[2]),
            (proc_i, proc_t, proc_o, KS[3]),
        ):
            nsub = k * sub_groups
            base = wid * SUB * nsub

            def step(s, _, idx_hbm=idx_hbm, tbl=tbl, out_hbm=out_hbm, base=base):
                off = base + s * SUB
                pltpu.sync_copy(idx_hbm.at[pl.ds(off, SUB)], idx_v)
                pltpu.async_copy(tbl.at[idx_v], rows_v, sem).wait()
                pltpu.sync_copy(rows_v, out_hbm.at[pl.ds(off, SUB)])
                return _

            lax.fori_loop(0, nsub, step, 0)

    return embed


def kernel(demographics_ints, vital_signs_ints, dx_ints, proc_ints,
           demo_table, vital_table, dx_table, proc_table, visit_table):
    batch_size = demographics_ints.shape[0]
    embed = _make_embed_call(batch_size)
    flat = [x.reshape(-1).astype(jnp.int32)
            for x in (demographics_ints, vital_signs_ints, dx_ints, proc_ints)]
    demo_f, vital_f, dx_f, proc_f = embed(
        flat[0], flat[1], flat[2], flat[3],
        demo_table, vital_table, dx_table, proc_table)
    demo_emb = demo_f.reshape(batch_size, KS[0], H)
    vital_emb = vital_f.reshape(batch_size, KS[1], H)
    dx_emb = dx_f.reshape(batch_size, KS[2], H)
    proc_emb = proc_f.reshape(batch_size, KS[3], H)
    visit_emb = jnp.broadcast_to(visit_table[None, :, :],
                                 (batch_size, 1, visit_table.shape[1]))
    mask_visit = jnp.ones((batch_size, 1), dtype=jnp.float32)
    mask_demo = jnp.ones((batch_size, KS[0]), dtype=jnp.float32)
    mask_vital = jnp.ones((batch_size, KS[1]), dtype=jnp.float32)
    return (demo_emb, vital_emb, dx_emb, proc_emb, visit_emb,
            mask_visit, mask_demo, mask_vital)


# speedup vs baseline: 3.5764x; 3.5764x over previous
import functools

import jax
import jax.numpy as jnp
from jax import lax
from jax.experimental import pallas as pl
from jax.experimental.pallas import tpu as pltpu
from jax.experimental.pallas import tpu_sc as plsc

H = 64
SUB = 128
KS = (9, 70, 200, 50)


@functools.lru_cache(maxsize=None)
def _make_embed_call(batch_size):
    info = plsc.get_sparse_core_info()
    nc, ns = info.num_cores, info.num_subcores
    nw = nc * ns
    rows_per_worker = batch_size // nw
    assert rows_per_worker % SUB == 0
    sub_groups = rows_per_worker // SUB

    mesh = plsc.VectorSubcoreMesh(core_axis_name="c", subcore_axis_name="s")

    out_type = tuple(
        jax.ShapeDtypeStruct((batch_size * k, H), jnp.float32) for k in KS
    )

    @functools.partial(
        pl.kernel,
        mesh=mesh,
        out_type=out_type,
        scratch_types=[
            pltpu.VMEM((SUB,), jnp.int32),
            pltpu.VMEM((SUB, H), jnp.float32),
            pltpu.SemaphoreType.DMA,
        ],
        compiler_params=pltpu.CompilerParams(use_tc_tiling_on_sc=False),
    )
    def embed(demo_i, vital_i, dx_i, proc_i,
              demo_t, vital_t, dx_t, proc_t,
              demo_o, vital_o, dx_o, proc_o,
              idx_v, rows_v, sem):
        wid = lax.axis_index("s") * nc + lax.axis_index("c")
        for (idx_hbm, tbl, out_hbm, k) in (
            (demo_i, demo_t, demo_o, KS[0]),
            (vital_i, vital_t, vital_o, KS[1]),
            (dx_i, dx_t, dx_o, KS[2]),
            (proc_i, proc_t, proc_o, KS[3]),
        ):
            nsub = k * sub_groups
            base = wid * SUB * nsub

            def step(s, _, idx_hbm=idx_hbm, tbl=tbl, out_hbm=out_hbm, base=base):
                off = base + s * SUB
                pltpu.sync_copy(idx_hbm.at[pl.ds(off, SUB)], idx_v)
                pltpu.async_copy(tbl.at[idx_v], rows_v, sem).wait()
                pltpu.sync_copy(rows_v, out_hbm.at[pl.ds(off, SUB)])
                return _

            lax.fori_loop(0, nsub, step, 0)

    return embed


def kernel(demographics_ints, vital_signs_ints, dx_ints, proc_ints,
           demo_table, vital_table, dx_table, proc_table, visit_table):
    batch_size = demographics_ints.shape[0]
    embed = _make_embed_call(batch_size)
    flat = [x.reshape(-1).astype(jnp.int32)
            for x in (demographics_ints, vital_signs_ints, dx_ints, proc_ints)]
    demo_f, vital_f, dx_f, proc_f = embed(
        flat[0], flat[1], flat[2], flat[3],
        demo_table, vital_table, dx_table, proc_table)
    demo_emb = demo_f.reshape(batch_size, KS[0], H)
    vital_emb = vital_f.reshape(batch_size, KS[1], H)
    dx_emb = dx_f.reshape(batch_size, KS[2], H)
    proc_emb = proc_f.reshape(batch_size, KS[3], H)
    visit_emb = jnp.broadcast_to(visit_table[None, :, :],
                                 (batch_size, 1, visit_table.shape[1]))
    mask_visit = jnp.ones((batch_size, 1), dtype=jnp.float32)
    mask_demo = jnp.ones((batch_size, KS[0]), dtype=jnp.float32)
    mask_vital = jnp.ones((batch_size, KS[1]), dtype=jnp.float32)
    return (demo_emb, vital_emb, dx_emb, proc_emb, visit_emb,
            mask_visit, mask_demo, mask_vital)


# speedup vs baseline: 4.5684x; 1.2774x over previous
import functools

import jax
import jax.numpy as jnp
from jax import lax
from jax.experimental import pallas as pl
from jax.experimental.pallas import tpu as pltpu
from jax.experimental.pallas import tpu_sc as plsc

H = 64
SUB = 128
KS = (9, 70, 200, 50)
NB = 4
NSUB_TOT = sum(KS)
F_OFF = (0, KS[0], KS[0] + KS[1], KS[0] + KS[1] + KS[2])


@functools.lru_cache(maxsize=None)
def _make_embed_call(batch_size):
    info = plsc.get_sparse_core_info()
    nc, ns = info.num_cores, info.num_subcores
    nw = nc * ns
    samples_per_worker = batch_size // nw
    assert samples_per_worker == SUB and batch_size == nw * SUB

    mesh = plsc.VectorSubcoreMesh(core_axis_name="c", subcore_axis_name="s")

    out_type = tuple(
        jax.ShapeDtypeStruct((batch_size * k, H), jnp.float32) for k in KS
    )

    @functools.partial(
        pl.kernel,
        mesh=mesh,
        out_type=out_type,
        scratch_types=[
            pltpu.VMEM((NSUB_TOT, SUB), jnp.int32),
            pltpu.VMEM((NB, SUB, H), jnp.float32),
            pltpu.SemaphoreType.DMA,
            pltpu.SemaphoreType.DMA((NB,)),
            pltpu.SemaphoreType.DMA((NB,)),
        ],
        compiler_params=pltpu.CompilerParams(use_tc_tiling_on_sc=False),
    )
    def embed(demo_i, vital_i, dx_i, proc_i,
              demo_t, vital_t, dx_t, proc_t,
              demo_o, vital_o, dx_o, proc_o,
              idx_all, rows, isem, gsem, ssem):
        wid = lax.axis_index("s") * nc + lax.axis_index("c")
        feats = (
            (demo_i, demo_t, demo_o, KS[0], F_OFF[0]),
            (vital_i, vital_t, vital_o, KS[1], F_OFF[1]),
            (dx_i, dx_t, dx_o, KS[2], F_OFF[2]),
            (proc_i, proc_t, proc_o, KS[3], F_OFF[3]),
        )
        handles = [
            pltpu.async_copy(idx_hbm.at[wid], idx_all.at[pl.ds(foff, k)], isem)
            for (idx_hbm, _, _, k, foff) in feats
        ]
        for h in handles:
            h.wait()

        for (_, tbl, out_hbm, k, foff) in feats:
            base = wid * SUB * k
            ngrp = (k + NB - 1) // NB

            def grp(g, carry, tbl=tbl, out_hbm=out_hbm, k=k, foff=foff,
                    base=base):
                for b in range(NB):
                    s = g * NB + b

                    @pl.when(jnp.logical_and(s < k, s >= NB))
                    def _(b=b):
                        pltpu.make_async_copy(
                            rows.at[b], out_hbm.at[pl.ds(0, SUB)],
                            ssem.at[b]).wait()

                    @pl.when(s < k)
                    def _(b=b, s=s):
                        pltpu.async_copy(
                            tbl.at[idx_all.at[foff + s]], rows.at[b],
                            gsem.at[b])
                for b in range(NB):
                    s = g * NB + b

                    @pl.when(s < k)
                    def _(b=b, s=s):
                        pltpu.make_async_copy(
                            tbl.at[pl.ds(0, SUB)], rows.at[b],
                            gsem.at[b]).wait()
                        pltpu.async_copy(
                            rows.at[b], out_hbm.at[pl.ds(base + s * SUB, SUB)],
                            ssem.at[b])
                return carry

            lax.fori_loop(0, ngrp, grp, 0)
            for b in range(NB):
                pltpu.make_async_copy(
                    rows.at[b], out_hbm.at[pl.ds(0, SUB)], ssem.at[b]).wait()

    return embed


def kernel(demographics_ints, vital_signs_ints, dx_ints, proc_ints,
           demo_table, vital_table, dx_table, proc_table, visit_table):
    batch_size = demographics_ints.shape[0]
    embed = _make_embed_call(batch_size)
    nw = batch_size // SUB
    flat = [x.astype(jnp.int32).reshape(nw, k, SUB)
            for x, k in zip((demographics_ints, vital_signs_ints,
                             dx_ints, proc_ints), KS)]
    demo_f, vital_f, dx_f, proc_f = embed(
        flat[0], flat[1], flat[2], flat[3],
        demo_table, vital_table, dx_table, proc_table)
    demo_emb = demo_f.reshape(batch_size, KS[0], H)
    vital_emb = vital_f.reshape(batch_size, KS[1], H)
    dx_emb = dx_f.reshape(batch_size, KS[2], H)
    proc_emb = proc_f.reshape(batch_size, KS[3], H)
    visit_emb = jnp.broadcast_to(visit_table[None, :, :],
                                 (batch_size, 1, visit_table.shape[1]))
    mask_visit = jnp.ones((batch_size, 1), dtype=jnp.float32)
    mask_demo = jnp.ones((batch_size, KS[0]), dtype=jnp.float32)
    mask_vital = jnp.ones((batch_size, KS[1]), dtype=jnp.float32)
    return (demo_emb, vital_emb, dx_emb, proc_emb, visit_emb,
            mask_visit, mask_demo, mask_vital)


# speedup vs baseline: 5.0628x; 1.1082x over previous
import functools

import jax
import jax.numpy as jnp
from jax import lax
from jax.experimental import pallas as pl
from jax.experimental.pallas import tpu as pltpu
from jax.experimental.pallas import tpu_sc as plsc

H = 64
SUB = 128
KS = (9, 70, 200, 50)
NB = 4


@functools.lru_cache(maxsize=None)
def _make_feature_call(batch_size, k, vocab):
    info = plsc.get_sparse_core_info()
    nc, ns = info.num_cores, info.num_subcores
    nw = nc * ns
    assert batch_size == nw * SUB

    mesh = plsc.VectorSubcoreMesh(core_axis_name="c", subcore_axis_name="s")

    @functools.partial(
        pl.kernel,
        mesh=mesh,
        out_type=jax.ShapeDtypeStruct((batch_size * k, H), jnp.float32),
        scratch_types=[
            pltpu.VMEM((k, SUB), jnp.int32),
            pltpu.VMEM((NB, SUB, H), jnp.float32),
            pltpu.SemaphoreType.DMA,
            pltpu.SemaphoreType.DMA((NB,)),
            pltpu.SemaphoreType.DMA((NB,)),
        ],
        compiler_params=pltpu.CompilerParams(use_tc_tiling_on_sc=False),
    )
    def embed(idx_t, tbl, out_hbm, idx_v, rows, isem, gsem, ssem):
        wid = lax.axis_index("s") * nc + lax.axis_index("c")
        base = wid * SUB * k
        pltpu.async_copy(
            idx_t.at[:, pl.ds(wid * SUB, SUB)], idx_v, isem).wait()

        ngrp = (k + NB - 1) // NB

        def grp(g, carry):
            for b in range(NB):
                s = g * NB + b

                @pl.when(jnp.logical_and(s < k, s >= NB))
                def _(b=b):
                    pltpu.make_async_copy(
                        rows.at[b], out_hbm.at[pl.ds(0, SUB)],
                        ssem.at[b]).wait()

                @pl.when(s < k)
                def _(b=b, s=s):
                    pltpu.async_copy(
                        tbl.at[idx_v.at[s]], rows.at[b], gsem.at[b])
            for b in range(NB):
                s = g * NB + b

                @pl.when(s < k)
                def _(b=b, s=s):
                    pltpu.make_async_copy(
                        tbl.at[pl.ds(0, SUB)], rows.at[b], gsem.at[b]).wait()
                    pltpu.async_copy(
                        rows.at[b], out_hbm.at[pl.ds(base + s * SUB, SUB)],
                        ssem.at[b])
            return carry

        lax.fori_loop(0, ngrp, grp, 0)
        for b in range(NB):
            pltpu.make_async_copy(
                rows.at[b], out_hbm.at[pl.ds(0, SUB)], ssem.at[b]).wait()

    return embed


def kernel(demographics_ints, vital_signs_ints, dx_ints, proc_ints,
           demo_table, vital_table, dx_table, proc_table, visit_table):
    batch_size = demographics_ints.shape[0]
    outs = []
    for ints, tbl, k in zip(
            (demographics_ints, vital_signs_ints, dx_ints, proc_ints),
            (demo_table, vital_table, dx_table, proc_table), KS):
        embed = _make_feature_call(batch_size, k, tbl.shape[0])
        flat = embed(ints.T.astype(jnp.int32), tbl)
        outs.append(flat.reshape(batch_size, k, H))
    demo_emb, vital_emb, dx_emb, proc_emb = outs
    visit_emb = jnp.broadcast_to(visit_table[None, :, :],
                                 (batch_size, 1, visit_table.shape[1]))
    mask_visit = jnp.ones((batch_size, 1), dtype=jnp.float32)
    mask_demo = jnp.ones((batch_size, KS[0]), dtype=jnp.float32)
    mask_vital = jnp.ones((batch_size, KS[1]), dtype=jnp.float32)
    return (demo_emb, vital_emb, dx_emb, proc_emb, visit_emb,
            mask_visit, mask_demo, mask_vital)
